# baseline (speedup 1.0000x reference)
import functools

import jax
import jax.numpy as jnp
from jax import lax
from jax.experimental import pallas as pl
from jax.experimental.pallas import tpu as pltpu
from jax.experimental.pallas import tpu_sc as plsc

VOCAB = 100000
DIM = 64
SEQ = 200
BATCH = 4096

NUM_CORES = 2
NUM_SUBCORES = 16
NUM_WORKERS = NUM_CORES * NUM_SUBCORES
BBLK = BATCH // NUM_WORKERS
LANES = 16
NBUF = 4
NIDX = 2 * NBUF


def _build_sc_kernel():
    mesh = plsc.VectorSubcoreMesh(core_axis_name="c", subcore_axis_name="s")

    @functools.partial(
        pl.kernel,
        mesh=mesh,
        compiler_params=pltpu.CompilerParams(use_tc_tiling_on_sc=False,
                                             needs_layout_passes=False),
        out_type=jax.ShapeDtypeStruct((SEQ * (DIM // 8), NUM_WORKERS, 1024),
                                      jnp.float32),
        scratch_types=[
            pltpu.VMEM((SEQ // 2, 2 * DIM), jnp.float32),
            [pltpu.VMEM((1, BBLK), jnp.int32)] * NIDX,
            [pltpu.VMEM((BBLK, DIM), jnp.float32)] * NBUF,
            [pltpu.VMEM((8, 1, 1024), jnp.float32)] * NBUF,
            [pltpu.SemaphoreType.DMA] * NIDX,
            [pltpu.SemaphoreType.DMA] * NBUF,
            [pltpu.SemaphoreType.DMA] * NBUF,
        ],
    )
    def k(ids_hbm, emb_hbm, pos_hbm, out_hbm, pos_vm, idxs, bufs, slabs,
          isems, gsems, wsems):
        wid = lax.axis_index("s") * NUM_CORES + lax.axis_index("c")
        pltpu.sync_copy(pos_hbm, pos_vm)

        kv = lax.iota(jnp.int32, 16)
        row_base = lax.shift_right_logical(kv, 1 + 1 + 1)
        col_base = lax.shift_left(kv & 7, 7)
        zero_v = kv & 0

        def start_idx(jj, s):
            pltpu.async_copy(ids_hbm.at[pl.ds(jj * NUM_WORKERS + wid, 1)],
                             idxs[s], isems[s])

        def wait_idx(jj, s):
            pltpu.make_async_copy(
                ids_hbm.at[pl.ds(jj * NUM_WORKERS + wid, 1)],
                idxs[s], isems[s]).wait()

        def start_gather(s, b):
            pltpu.async_copy(emb_hbm.at[idxs[s].at[0]], bufs[b], gsems[b])

        def wait_gather(s, b):
            pltpu.make_async_copy(emb_hbm.at[idxs[s].at[0]], bufs[b],
                                  gsems[b]).wait()

        def out_slice(jj):
            return out_hbm.at[pl.ds(jj * (DIM // 8), DIM // 8),
                              pl.ds(wid, 1), :]

        for s in range(NIDX):
            start_idx(s, s)
        for b in range(NBUF):
            wait_idx(b, b)
            start_gather(b, b)

        @pl.loop(0, SEQ, step=NIDX)
        def _(j):
            for bb in range(NIDX):
                jj = j + bb
                b = bb % NBUF
                wait_gather(bb, b)

                @pl.when(jj >= NBUF)
                def _():
                    pltpu.make_async_copy(slabs[b], out_slice(jj - NBUF),
                                          wsems[b]).wait()

                lh = jj // 2
                lc0 = (jj % 2) * DIM
                pvec = [pos_vm.at[lh, pl.ds(lc0 + c * LANES, LANES)][...]
                        for c in range(4)]
                dt = [row_base + 2 * c for c in range(4)]

                @pl.loop(0, BBLK)
                def _(t):
                    col = col_base + t
                    for c in range(4):
                        val = bufs[b].at[t, pl.ds(c * LANES, LANES)][...]
                        plsc.store_scatter(slabs[b], [dt[c], zero_v, col],
                                           val + pvec[c])

                pltpu.async_copy(slabs[b], out_slice(jj), wsems[b])

                @pl.when(jj + NBUF < SEQ)
                def _():
                    s = (bb + NBUF) % NIDX
                    wait_idx(jj + NBUF, s)
                    start_gather(s, b)

                @pl.when(jj + NIDX < SEQ)
                def _():
                    start_idx(jj + NIDX, bb)

        for b in range(NBUF):
            jj = SEQ - NBUF + b
            pltpu.make_async_copy(slabs[b], out_slice(jj), wsems[b]).wait()

    return k


_sc_kernel = _build_sc_kernel()


def kernel(token_ids, emb_table, pos_table):
    ids = token_ids.astype(jnp.int32).T.reshape(SEQ * NUM_WORKERS, BBLK)
    pos = pos_table[:SEQ].reshape(SEQ // 2, 2 * DIM)
    out = _sc_kernel(ids, emb_table, pos)
    out = out.reshape(SEQ, DIM // 8, NUM_WORKERS, 8, BBLK)
    out = out.transpose(2, 4, 0, 1, 3).reshape(BATCH, SEQ, DIM)
    return out


# speedup vs baseline: 1.0188x; 1.0188x over previous
import functools

import jax
import jax.numpy as jnp
from jax import lax
from jax.experimental import pallas as pl
from jax.experimental.pallas import tpu as pltpu
from jax.experimental.pallas import tpu_sc as plsc

VOCAB = 100000
DIM = 64
SEQ = 200
BATCH = 4096

NUM_CORES = 2
NUM_SUBCORES = 16
NUM_WORKERS = NUM_CORES * NUM_SUBCORES
BBLK = BATCH // NUM_WORKERS
LANES = 16
NBUF = 4
NIDX = 2 * NBUF


def _build_sc_kernel():
    mesh = plsc.VectorSubcoreMesh(core_axis_name="c", subcore_axis_name="s")

    @functools.partial(
        pl.kernel,
        mesh=mesh,
        compiler_params=pltpu.CompilerParams(use_tc_tiling_on_sc=False,
                                             needs_layout_passes=False),
        out_type=jax.ShapeDtypeStruct((SEQ * (DIM // 8), NUM_WORKERS, 1024),
                                      jnp.float32),
        scratch_types=[
            pltpu.VMEM((SEQ // 2, 2 * DIM), jnp.float32),
            [pltpu.VMEM((1, BBLK), jnp.int32)] * NIDX,
            [pltpu.VMEM((BBLK, DIM), jnp.float32)] * NBUF,
            [pltpu.VMEM((8, 1, 1024), jnp.float32)] * NBUF,
            [pltpu.SemaphoreType.DMA] * NIDX,
            [pltpu.SemaphoreType.DMA] * NBUF,
            [pltpu.SemaphoreType.DMA] * NBUF,
        ],
    )
    def k(ids_hbm, emb_hbm, pos_hbm, out_hbm, pos_vm, idxs, bufs, slabs,
          isems, gsems, wsems):
        wid = lax.axis_index("s") * NUM_CORES + lax.axis_index("c")
        pltpu.sync_copy(pos_hbm, pos_vm)

        kv = lax.iota(jnp.int32, 16)
        row_base = lax.shift_right_logical(kv, 1 + 1 + 1)
        col_base = lax.shift_left(kv & 7, 7)
        zero_v = kv & 0

        def start_idx(jj, s):
            pltpu.async_copy(ids_hbm.at[pl.ds(jj * NUM_WORKERS + wid, 1)],
                             idxs[s], isems[s])

        def wait_idx(jj, s):
            pltpu.make_async_copy(
                ids_hbm.at[pl.ds(jj * NUM_WORKERS + wid, 1)],
                idxs[s], isems[s]).wait()

        def start_gather(s, b):
            pltpu.async_copy(emb_hbm.at[idxs[s].at[0]], bufs[b], gsems[b])

        def wait_gather(s, b):
            pltpu.make_async_copy(emb_hbm.at[idxs[s].at[0]], bufs[b],
                                  gsems[b]).wait()

        def out_slice(jj):
            return out_hbm.at[pl.ds(jj * (DIM // 8), DIM // 8),
                              pl.ds(wid, 1), :]

        for s in range(NIDX):
            start_idx(s, s)
        for b in range(NBUF):
            wait_idx(b, b)
            start_gather(b, b)

        @pl.loop(0, SEQ, step=NIDX)
        def _(j):
            for bb in range(NIDX):
                jj = j + bb
                b = bb % NBUF
                wait_gather(bb, b)

                @pl.when(jj >= NBUF)
                def _():
                    pltpu.make_async_copy(slabs[b], out_slice(jj - NBUF),
                                          wsems[b]).wait()

                lh = jj // 2
                lc0 = (jj % 2) * DIM
                pvec = [pos_vm.at[lh, pl.ds(lc0 + c * LANES, LANES)][...]
                        for c in range(4)]
                dt = [row_base + 2 * c for c in range(4)]

                @pl.loop(0, BBLK, step=4)
                def _(t):
                    for tt in range(4):
                        col = col_base + (t + tt)
                        for c in range(4):
                            val = bufs[b].at[t + tt,
                                             pl.ds(c * LANES, LANES)][...]
                            plsc.store_scatter(slabs[b],
                                               [dt[c], zero_v, col],
                                               val + pvec[c])

                pltpu.async_copy(slabs[b], out_slice(jj), wsems[b])

                @pl.when(jj + NBUF < SEQ)
                def _():
                    s = (bb + NBUF) % NIDX
                    wait_idx(jj + NBUF, s)
                    start_gather(s, b)

                @pl.when(jj + NIDX < SEQ)
                def _():
                    start_idx(jj + NIDX, bb)

        for b in range(NBUF):
            jj = SEQ - NBUF + b
            pltpu.make_async_copy(slabs[b], out_slice(jj), wsems[b]).wait()

    return k


_sc_kernel = _build_sc_kernel()


def kernel(token_ids, emb_table, pos_table):
    ids = token_ids.astype(jnp.int32).T.reshape(SEQ * NUM_WORKERS, BBLK)
    pos = pos_table[:SEQ].reshape(SEQ // 2, 2 * DIM)
    out = _sc_kernel(ids, emb_table, pos)
    out = out.reshape(SEQ, DIM // 8, NUM_WORKERS, 8, BBLK)
    out = out.transpose(2, 4, 0, 1, 3).reshape(BATCH, SEQ, DIM)
    return out


# speedup vs baseline: 2.0674x; 2.0293x over previous
import functools

import jax
import jax.numpy as jnp
from jax import lax
from jax.experimental import pallas as pl
from jax.experimental.pallas import tpu as pltpu
from jax.experimental.pallas import tpu_sc as plsc

VOCAB = 100000
DIM = 64
SEQ = 200
BATCH = 4096

NUM_CORES = 2
NUM_SUBCORES = 16
NUM_WORKERS = NUM_CORES * NUM_SUBCORES
BBLK = BATCH // NUM_WORKERS
LANES = 16
NBUF = 4
NIDX = 2 * NBUF


def _build_sc_kernel():
    mesh = plsc.VectorSubcoreMesh(core_axis_name="c", subcore_axis_name="s")

    @functools.partial(
        pl.kernel,
        mesh=mesh,
        compiler_params=pltpu.CompilerParams(use_tc_tiling_on_sc=False,
                                             needs_layout_passes=False),
        out_type=jax.ShapeDtypeStruct((SEQ * (DIM // 8), NUM_WORKERS, 8, 128),
                                      jnp.float32),
        scratch_types=[
            pltpu.VMEM((SEQ // 2, 2 * DIM), jnp.float32),
            [pltpu.VMEM((1, BBLK), jnp.int32)] * NIDX,
            [pltpu.VMEM((BBLK, DIM), jnp.float32)] * NBUF,
            [pltpu.VMEM((8, 1, 8, 129), jnp.float32)] * NBUF,
            [pltpu.SemaphoreType.DMA] * NIDX,
            [pltpu.SemaphoreType.DMA] * NBUF,
            [pltpu.SemaphoreType.DMA] * NBUF,
        ],
    )
    def k(ids_hbm, emb_hbm, pos_hbm, out_hbm, pos_vm, idxs, bufs, slabs,
          isems, gsems, wsems):
        wid = lax.axis_index("s") * NUM_CORES + lax.axis_index("c")
        pltpu.sync_copy(pos_hbm, pos_vm)

        kv = lax.iota(jnp.int32, 16)
        row_base = lax.shift_right_logical(kv, 1 + 1 + 1)
        dp_v = kv & 7
        zero_v = kv & 0

        def start_idx(jj, s):
            pltpu.async_copy(ids_hbm.at[pl.ds(jj * NUM_WORKERS + wid, 1)],
                             idxs[s], isems[s])

        def wait_idx(jj, s):
            pltpu.make_async_copy(
                ids_hbm.at[pl.ds(jj * NUM_WORKERS + wid, 1)],
                idxs[s], isems[s]).wait()

        def start_gather(s, b):
            pltpu.async_copy(emb_hbm.at[idxs[s].at[0]], bufs[b], gsems[b])

        def wait_gather(s, b):
            pltpu.make_async_copy(emb_hbm.at[idxs[s].at[0]], bufs[b],
                                  gsems[b]).wait()

        def out_slice(jj):
            return out_hbm.at[pl.ds(jj * (DIM // 8), DIM // 8),
                              pl.ds(wid, 1), :, :]

        def slab_src(b):
            return slabs[b].at[:, :, :, pl.ds(0, 128)]

        for s in range(NIDX):
            start_idx(s, s)
        for b in range(NBUF):
            wait_idx(b, b)
            start_gather(b, b)

        @pl.loop(0, SEQ, step=NIDX)
        def _(j):
            for bb in range(NIDX):
                jj = j + bb
                b = bb % NBUF
                wait_gather(bb, b)

                @pl.when(jj >= NBUF)
                def _():
                    pltpu.make_async_copy(slab_src(b), out_slice(jj - NBUF),
                                          wsems[b]).wait()

                lh = jj // 2
                lc0 = (jj % 2) * DIM
                pvec = [pos_vm.at[lh, pl.ds(lc0 + c * LANES, LANES)][...]
                        for c in range(4)]
                dt = [row_base + 2 * c for c in range(4)]

                @pl.loop(0, BBLK, step=4)
                def _(t):
                    for tt in range(4):
                        t_v = zero_v + (t + tt)
                        for c in range(4):
                            val = bufs[b].at[t + tt,
                                             pl.ds(c * LANES, LANES)][...]
                            plsc.store_scatter(slabs[b],
                                               [dt[c], zero_v, dp_v, t_v],
                                               val + pvec[c])

                pltpu.async_copy(slab_src(b), out_slice(jj), wsems[b])

                @pl.when(jj + NBUF < SEQ)
                def _():
                    s = (bb + NBUF) % NIDX
                    wait_idx(jj + NBUF, s)
                    start_gather(s, b)

                @pl.when(jj + NIDX < SEQ)
                def _():
                    start_idx(jj + NIDX, bb)

        for b in range(NBUF):
            jj = SEQ - NBUF + b
            pltpu.make_async_copy(slab_src(b), out_slice(jj),
                                  wsems[b]).wait()

    return k


_sc_kernel = _build_sc_kernel()


def kernel(token_ids, emb_table, pos_table):
    ids = token_ids.astype(jnp.int32).T.reshape(SEQ * NUM_WORKERS, BBLK)
    pos = pos_table[:SEQ].reshape(SEQ // 2, 2 * DIM)
    out = _sc_kernel(ids, emb_table, pos)
    out = out.reshape(SEQ, DIM // 8, NUM_WORKERS, 8, BBLK)
    out = out.transpose(2, 4, 0, 1, 3).reshape(BATCH, SEQ, DIM)
    return out


# speedup vs baseline: 3.5212x; 1.7032x over previous
import functools

import jax
import jax.numpy as jnp
from jax import lax
from jax.experimental import pallas as pl
from jax.experimental.pallas import tpu as pltpu
from jax.experimental.pallas import tpu_sc as plsc

VOCAB = 100000
DIM = 64
SEQ = 200
BATCH = 4096

NUM_CORES = 2
NUM_SUBCORES = 16
NUM_WORKERS = NUM_CORES * NUM_SUBCORES
BBLK = BATCH // NUM_WORKERS
LANES = 16
NBUF = 4
NIDX = 2 * NBUF


def _build_sc_kernel():
    mesh = plsc.VectorSubcoreMesh(core_axis_name="c", subcore_axis_name="s")

    @functools.partial(
        pl.kernel,
        mesh=mesh,
        compiler_params=pltpu.CompilerParams(use_tc_tiling_on_sc=False,
                                             needs_layout_passes=False),
        out_type=jax.ShapeDtypeStruct((SEQ * (DIM // 8), NUM_WORKERS, 8, 128),
                                      jnp.float32),
        scratch_types=[
            pltpu.VMEM((SEQ // 2, 2 * DIM), jnp.float32),
            [pltpu.VMEM((1, BBLK), jnp.int32)] * NIDX,
            [pltpu.VMEM((BBLK, DIM), jnp.float32)] * NBUF,
            [pltpu.VMEM((8, 1, 8, 129), jnp.float32)] * NBUF,
            [pltpu.SemaphoreType.DMA] * NIDX,
            [pltpu.SemaphoreType.DMA] * NBUF,
            [pltpu.SemaphoreType.DMA] * NBUF,
        ],
    )
    def k(ids_hbm, emb_hbm, pos_hbm, out_hbm, pos_vm, idxs, bufs, slabs,
          isems, gsems, wsems):
        wid = lax.axis_index("s") * NUM_CORES + lax.axis_index("c")
        pltpu.sync_copy(pos_hbm, pos_vm)

        kv = lax.iota(jnp.int32, 16)
        row_base = lax.shift_right_logical(kv, 1 + 1 + 1)
        dp_v = kv & 7
        zero_v = kv & 0

        def start_idx(jj, s):
            pltpu.async_copy(ids_hbm.at[pl.ds(jj * NUM_WORKERS + wid, 1)],
                             idxs[s], isems[s])

        def wait_idx(jj, s):
            pltpu.make_async_copy(
                ids_hbm.at[pl.ds(jj * NUM_WORKERS + wid, 1)],
                idxs[s], isems[s]).wait()

        def start_gather(s, b):
            pltpu.async_copy(emb_hbm.at[idxs[s].at[0]], bufs[b], gsems[b])

        def wait_gather(s, b):
            pltpu.make_async_copy(emb_hbm.at[idxs[s].at[0]], bufs[b],
                                  gsems[b]).wait()

        def out_slice(jj):
            return out_hbm.at[pl.ds(jj * (DIM // 8), DIM // 8),
                              pl.ds(wid, 1), :, :]

        def slab_src(b):
            return slabs[b].at[:, :, :, pl.ds(0, 128)]

        for s in range(NIDX):
            start_idx(s, s)
        for b in range(NBUF):
            wait_idx(b, b)
            start_gather(b, b)

        @pl.loop(0, SEQ, step=NIDX)
        def _(j):
            for bb in range(NIDX):
                jj = j + bb
                b = bb % NBUF
                wait_gather(bb, b)

                @pl.when(jj >= NBUF)
                def _():
                    pltpu.make_async_copy(slab_src(b), out_slice(jj - NBUF),
                                          wsems[b]).wait()

                lh = jj // 2
                lc0 = (jj % 2) * DIM
                pvec = [pos_vm.at[lh, pl.ds(lc0 + c * LANES, LANES)][...]
                        for c in range(4)]
                dt = [row_base + 2 * c for c in range(4)]

                @pl.loop(0, BBLK, step=2)
                def _(t):
                    vals = []
                    for tt in range(2):
                        for c in range(4):
                            v = bufs[b].at[t + tt,
                                           pl.ds(c * LANES, LANES)][...]
                            vals.append(v + pvec[c])
                    for tt in range(2):
                        t_v = zero_v + (t + tt)
                        for c in range(4):
                            plsc.store_scatter(slabs[b],
                                               [dt[c], zero_v, dp_v, t_v],
                                               vals[tt * 4 + c])

                pltpu.async_copy(slab_src(b), out_slice(jj), wsems[b])

                @pl.when(jj + NBUF < SEQ)
                def _():
                    s = (bb + NBUF) % NIDX
                    wait_idx(jj + NBUF, s)
                    start_gather(s, b)

                @pl.when(jj + NIDX < SEQ)
                def _():
                    start_idx(jj + NIDX, bb)

        for b in range(NBUF):
            jj = SEQ - NBUF + b
            pltpu.make_async_copy(slab_src(b), out_slice(jj),
                                  wsems[b]).wait()

    return k


_sc_kernel = _build_sc_kernel()


def kernel(token_ids, emb_table, pos_table):
    ids = token_ids.astype(jnp.int32).T.reshape(SEQ * NUM_WORKERS, BBLK)
    pos = pos_table[:SEQ].reshape(SEQ // 2, 2 * DIM)
    out = _sc_kernel(ids, emb_table, pos)
    out = out.reshape(SEQ, DIM // 8, NUM_WORKERS, 8, BBLK)
    out = out.transpose(2, 4, 0, 1, 3).reshape(BATCH, SEQ, DIM)
    return out
